# Initial kernel scaffold; baseline (speedup 1.0000x reference)
import jax
import jax.numpy as jnp
from jax.experimental import pallas as pl


def kernel(x, e1_Wih, e1_Whh, e1_bih, e1_bhh, e2_Wih, e2_Whh, e2_bih, e2_bhh, d1_Wih, d1_Whh, d1_bih, d1_bhh, d2_Wih, d2_Whh, d2_bih, d2_bhh, out_W, out_b):
    raise NotImplementedError("write your pallas kernel here")


# speedup vs baseline: 4.5156x; 4.5156x over previous
import jax
import jax.numpy as jnp
from jax.experimental import pallas as pl
from jax.experimental.pallas import tpu as pltpu

_VMEM_LIMIT = 56 * 1024 * 1024


def _gates(xg, hg, h, H):
    r = jax.nn.sigmoid(xg[:, :H] + hg[:, :H])
    z = jax.nn.sigmoid(xg[:, H:2 * H] + hg[:, H:2 * H])
    n = jnp.tanh(xg[:, 2 * H:] + r * hg[:, 2 * H:])
    return (1.0 - z) * n + z * h


def _e1_body(tb, hid):
    def body(x_ref, wih_ref, bih_ref, bhh_ref, whh_hbm, ys_ref, whh_v, h_ref, sem):
        @pl.when(pl.program_id(0) == 0)
        def _():
            cp = pltpu.make_async_copy(whh_hbm, whh_v, sem)
            cp.start()
            cp.wait()
            h_ref[...] = jnp.zeros_like(h_ref)

        def step(t, c):
            xg = x_ref[pl.ds(t, 1), :] * wih_ref[...] + bih_ref[...]
            hb = h_ref[...].astype(jnp.bfloat16)
            hg = jnp.dot(hb, whh_v[...],
                         preferred_element_type=jnp.float32) + bhh_ref[...]
            h_new = _gates(xg, hg, h_ref[...], hid)
            h_ref[...] = h_new
            ys_ref[pl.ds(t, 1), :] = h_new
            return c

        jax.lax.fori_loop(0, tb, step, 0)
    return body


def _run_e1(x, wih_row, bih, bhh, whh_t, *, tb, interpret=False):
    seq = x.shape[0]
    hid = whh_t.shape[0]
    return pl.pallas_call(
        _e1_body(tb, hid),
        grid=(seq // tb,),
        in_specs=[
            pl.BlockSpec((tb, 1), lambda i: (i, 0)),
            pl.BlockSpec((1, 3 * hid), lambda i: (0, 0)),
            pl.BlockSpec((1, 3 * hid), lambda i: (0, 0)),
            pl.BlockSpec((1, 3 * hid), lambda i: (0, 0)),
            pl.BlockSpec(memory_space=pl.ANY),
        ],
        out_specs=pl.BlockSpec((tb, hid), lambda i: (i, 0)),
        out_shape=jax.ShapeDtypeStruct((seq, hid), jnp.float32),
        scratch_shapes=[
            pltpu.VMEM((hid, 3 * hid), jnp.bfloat16),
            pltpu.VMEM((1, hid), jnp.float32),
            pltpu.SemaphoreType.DMA,
        ],
        compiler_params=pltpu.CompilerParams(
            dimension_semantics=("arbitrary",),
            vmem_limit_bytes=_VMEM_LIMIT,
        ),
        name="gru_e1",
        interpret=interpret,
    )(x, wih_row, bih, bhh, whh_t)


def _e2_body(tb, emb):
    def body(ys1_ref, bih_ref, bhh_ref, wih_hbm, whh_hbm, ht_ref,
             wih_v, whh_v, xg_scr, h_ref, sem1, sem2):
        @pl.when(pl.program_id(0) == 0)
        def _():
            cp1 = pltpu.make_async_copy(wih_hbm, wih_v, sem1)
            cp2 = pltpu.make_async_copy(whh_hbm, whh_v, sem2)
            cp1.start()
            cp2.start()
            cp1.wait()
            cp2.wait()
            h_ref[...] = jnp.zeros_like(h_ref)

        xg_scr[...] = jnp.dot(ys1_ref[...].astype(jnp.bfloat16), wih_v[...],
                              preferred_element_type=jnp.float32) + bih_ref[...]

        def step(t, c):
            hb = h_ref[...].astype(jnp.bfloat16)
            hg = jnp.dot(hb, whh_v[...],
                         preferred_element_type=jnp.float32) + bhh_ref[...]
            h_ref[...] = _gates(xg_scr[pl.ds(t, 1), :], hg, h_ref[...], emb)
            return c

        jax.lax.fori_loop(0, tb, step, 0)
        ht_ref[...] = h_ref[...]
    return body


def _run_e2(ys1, bih, bhh, wih_t, whh_t, *, tb, interpret=False):
    seq, hid = ys1.shape
    emb = whh_t.shape[0]
    return pl.pallas_call(
        _e2_body(tb, emb),
        grid=(seq // tb,),
        in_specs=[
            pl.BlockSpec((tb, hid), lambda i: (i, 0)),
            pl.BlockSpec((1, 3 * emb), lambda i: (0, 0)),
            pl.BlockSpec((1, 3 * emb), lambda i: (0, 0)),
            pl.BlockSpec(memory_space=pl.ANY),
            pl.BlockSpec(memory_space=pl.ANY),
        ],
        out_specs=pl.BlockSpec((1, emb), lambda i: (0, 0)),
        out_shape=jax.ShapeDtypeStruct((1, emb), jnp.float32),
        scratch_shapes=[
            pltpu.VMEM((hid, 3 * emb), jnp.bfloat16),
            pltpu.VMEM((emb, 3 * emb), jnp.bfloat16),
            pltpu.VMEM((tb, 3 * emb), jnp.float32),
            pltpu.VMEM((1, emb), jnp.float32),
            pltpu.SemaphoreType.DMA,
            pltpu.SemaphoreType.DMA,
        ],
        compiler_params=pltpu.CompilerParams(
            dimension_semantics=("arbitrary",),
            vmem_limit_bytes=_VMEM_LIMIT,
        ),
        name="gru_e2",
        interpret=interpret,
    )(ys1, bih, bhh, wih_t, whh_t)


def _d1_body(tb, emb):
    def body(emb_ref, bih_ref, bhh_ref, wih_hbm, whh_hbm, ys_ref,
             wih_v, whh_v, xg_ref, h_ref, sem1, sem2):
        @pl.when(pl.program_id(0) == 0)
        def _():
            cp1 = pltpu.make_async_copy(wih_hbm, wih_v, sem1)
            cp2 = pltpu.make_async_copy(whh_hbm, whh_v, sem2)
            cp1.start()
            cp2.start()
            cp1.wait()
            cp2.wait()
            h_ref[...] = jnp.zeros_like(h_ref)
            xg_ref[...] = jnp.dot(emb_ref[...].astype(jnp.bfloat16), wih_v[...],
                                  preferred_element_type=jnp.float32) + bih_ref[...]

        def step(t, c):
            hb = h_ref[...].astype(jnp.bfloat16)
            hg = jnp.dot(hb, whh_v[...],
                         preferred_element_type=jnp.float32) + bhh_ref[...]
            h_new = _gates(xg_ref[...], hg, h_ref[...], emb)
            h_ref[...] = h_new
            ys_ref[pl.ds(t, 1), :] = h_new
            return c

        jax.lax.fori_loop(0, tb, step, 0)
    return body


def _run_d1(emb_vec, bih, bhh, wih_t, whh_t, *, seq, tb, interpret=False):
    emb = whh_t.shape[0]
    return pl.pallas_call(
        _d1_body(tb, emb),
        grid=(seq // tb,),
        in_specs=[
            pl.BlockSpec((1, emb), lambda i: (0, 0)),
            pl.BlockSpec((1, 3 * emb), lambda i: (0, 0)),
            pl.BlockSpec((1, 3 * emb), lambda i: (0, 0)),
            pl.BlockSpec(memory_space=pl.ANY),
            pl.BlockSpec(memory_space=pl.ANY),
        ],
        out_specs=pl.BlockSpec((tb, emb), lambda i: (i, 0)),
        out_shape=jax.ShapeDtypeStruct((seq, emb), jnp.float32),
        scratch_shapes=[
            pltpu.VMEM((emb, 3 * emb), jnp.bfloat16),
            pltpu.VMEM((emb, 3 * emb), jnp.bfloat16),
            pltpu.VMEM((1, 3 * emb), jnp.float32),
            pltpu.VMEM((1, emb), jnp.float32),
            pltpu.SemaphoreType.DMA,
            pltpu.SemaphoreType.DMA,
        ],
        compiler_params=pltpu.CompilerParams(
            dimension_semantics=("arbitrary",),
            vmem_limit_bytes=_VMEM_LIMIT,
        ),
        name="gru_d1",
        interpret=interpret,
    )(emb_vec, bih, bhh, wih_t, whh_t)


def _d2_body(tb, hid):
    def body(ys2_ref, bih_ref, bhh_ref, outw_ref, outb_ref, wih_hbm, whh_hbm,
             y_ref, wih_v, whh_v, xg_scr, ys3_scr, h_ref, sem1, sem2):
        @pl.when(pl.program_id(0) == 0)
        def _():
            cp1 = pltpu.make_async_copy(wih_hbm, wih_v, sem1)
            cp2 = pltpu.make_async_copy(whh_hbm, whh_v, sem2)
            cp1.start()
            cp2.start()
            cp1.wait()
            cp2.wait()
            h_ref[...] = jnp.zeros_like(h_ref)

        xg_scr[...] = jnp.dot(ys2_ref[...].astype(jnp.bfloat16), wih_v[...],
                              preferred_element_type=jnp.float32) + bih_ref[...]

        def step(t, c):
            hb = h_ref[...].astype(jnp.bfloat16)
            hg = jnp.dot(hb, whh_v[...],
                         preferred_element_type=jnp.float32) + bhh_ref[...]
            h_new = _gates(xg_scr[pl.ds(t, 1), :], hg, h_ref[...], hid)
            h_ref[...] = h_new
            ys3_scr[pl.ds(t, 1), :] = h_new
            return c

        jax.lax.fori_loop(0, tb, step, 0)
        y_ref[...] = jnp.dot(ys3_scr[...].astype(jnp.bfloat16), outw_ref[...],
                             preferred_element_type=jnp.float32) + outb_ref[...]
    return body


def _run_d2(ys2, bih, bhh, out_wt, out_b, wih_t, whh_t, *, tb, interpret=False):
    seq, emb = ys2.shape
    hid = whh_t.shape[0]
    return pl.pallas_call(
        _d2_body(tb, hid),
        grid=(seq // tb,),
        in_specs=[
            pl.BlockSpec((tb, emb), lambda i: (i, 0)),
            pl.BlockSpec((1, 3 * hid), lambda i: (0, 0)),
            pl.BlockSpec((1, 3 * hid), lambda i: (0, 0)),
            pl.BlockSpec((hid, 1), lambda i: (0, 0)),
            pl.BlockSpec((1, 1), lambda i: (0, 0)),
            pl.BlockSpec(memory_space=pl.ANY),
            pl.BlockSpec(memory_space=pl.ANY),
        ],
        out_specs=pl.BlockSpec((tb, 1), lambda i: (i, 0)),
        out_shape=jax.ShapeDtypeStruct((seq, 1), jnp.float32),
        scratch_shapes=[
            pltpu.VMEM((emb, 3 * hid), jnp.bfloat16),
            pltpu.VMEM((hid, 3 * hid), jnp.bfloat16),
            pltpu.VMEM((tb, 3 * hid), jnp.float32),
            pltpu.VMEM((tb, hid), jnp.float32),
            pltpu.VMEM((1, hid), jnp.float32),
            pltpu.SemaphoreType.DMA,
            pltpu.SemaphoreType.DMA,
        ],
        compiler_params=pltpu.CompilerParams(
            dimension_semantics=("arbitrary",),
            vmem_limit_bytes=_VMEM_LIMIT,
        ),
        name="gru_d2",
        interpret=interpret,
    )(ys2, bih, bhh, out_wt, out_b, wih_t, whh_t)


def _gruae(x, e1_Wih, e1_Whh, e1_bih, e1_bhh,
           e2_Wih, e2_Whh, e2_bih, e2_bhh,
           d1_Wih, d1_Whh, d1_bih, d1_bhh,
           d2_Wih, d2_Whh, d2_bih, d2_bhh,
           out_W, out_b, *, interpret=False):
    bf16 = jnp.bfloat16
    seq = x.shape[0]
    hid = e1_Whh.shape[1]
    emb = e2_Whh.shape[1]
    tb = min(512, seq)
    tb2 = min(256, seq)

    ys1 = _run_e1(
        x.reshape(seq, 1),
        e1_Wih.reshape(1, 3 * hid) if e1_Wih.shape == (3 * hid, 1) else e1_Wih.T,
        e1_bih.reshape(1, -1), e1_bhh.reshape(1, -1),
        e1_Whh.T.astype(bf16),
        tb=tb, interpret=interpret)

    ht = _run_e2(
        ys1, e2_bih.reshape(1, -1), e2_bhh.reshape(1, -1),
        e2_Wih.T.astype(bf16), e2_Whh.T.astype(bf16),
        tb=tb, interpret=interpret)

    ys2 = _run_d1(
        ht, d1_bih.reshape(1, -1), d1_bhh.reshape(1, -1),
        d1_Wih.T.astype(bf16), d1_Whh.T.astype(bf16),
        seq=seq, tb=tb, interpret=interpret)

    y = _run_d2(
        ys2, d2_bih.reshape(1, -1), d2_bhh.reshape(1, -1),
        out_W.T.astype(bf16), out_b.reshape(1, 1),
        d2_Wih.T.astype(bf16), d2_Whh.T.astype(bf16),
        tb=tb2, interpret=interpret)

    return y


def kernel(x, e1_Wih, e1_Whh, e1_bih, e1_bhh,
           e2_Wih, e2_Whh, e2_bih, e2_bhh,
           d1_Wih, d1_Whh, d1_bih, d1_bhh,
           d2_Wih, d2_Whh, d2_bih, d2_bhh,
           out_W, out_b):
    return _gruae(x, e1_Wih, e1_Whh, e1_bih, e1_bhh,
                  e2_Wih, e2_Whh, e2_bih, e2_bhh,
                  d1_Wih, d1_Whh, d1_bih, d1_bhh,
                  d2_Wih, d2_Whh, d2_bih, d2_bhh,
                  out_W, out_b)


# speedup vs baseline: 4.7496x; 1.0518x over previous
import functools

import jax
import jax.numpy as jnp
from jax.experimental import pallas as pl
from jax.experimental.pallas import tpu as pltpu
from jax.sharding import Mesh, PartitionSpec as P
import numpy as np

_VMEM_LIMIT = 56 * 1024 * 1024


def _gates(xg, hg, h, hh):
    r = jax.nn.sigmoid(xg[:, :hh] + hg[:, :hh])
    z = jax.nn.sigmoid(xg[:, hh:2 * hh] + hg[:, hh:2 * hh])
    n = jnp.tanh(xg[:, 2 * hh:] + r * hg[:, 2 * hh:])
    return (1.0 - z) * n + z * h


def _rcopy(send_buf, recv_buf, slot, send_sem, recv_sem, partner):
    return pltpu.make_async_remote_copy(
        send_buf.at[slot], recv_buf.at[slot], send_sem, recv_sem,
        device_id=partner, device_id_type=pltpu.DeviceIdType.LOGICAL)


def _step(s, xg_t, h_ref, w_own, w_oth, bhh_ref, send_buf, recv_buf,
          send_sem, recv_sem, partner, hh):
    slot = jnp.bitwise_and(s, 1)
    slot_prev = jnp.bitwise_and(s + 1, 1)

    hb = h_ref[...].astype(jnp.bfloat16)
    hg_a = jnp.dot(hb, w_own[...], preferred_element_type=jnp.float32)

    @pl.when(s > 0)
    def _():
        _rcopy(send_buf, recv_buf, slot_prev, send_sem, recv_sem,
               partner).wait_recv()

    h_oth = recv_buf[slot_prev].astype(jnp.bfloat16)
    hg_b = jnp.dot(h_oth, w_oth[...], preferred_element_type=jnp.float32)
    hg = hg_a + hg_b + bhh_ref[...]
    h_new = _gates(xg_t, hg, h_ref[...], hh)
    h_ref[...] = h_new

    @pl.when(s >= 2)
    def _():
        _rcopy(send_buf, recv_buf, slot, send_sem, recv_sem,
               partner).wait_send()

    send_buf[slot] = h_new
    _rcopy(send_buf, recv_buf, slot, send_sem, recv_sem, partner).start()
    return h_new


def _prologue(i, partner, h_ref, recv_buf, dma_pairs):
    @pl.when(i == 0)
    def _():
        copies = [pltpu.make_async_copy(src, dst, sem)
                  for src, dst, sem in dma_pairs]
        for cp in copies:
            cp.start()
        h_ref[...] = jnp.zeros_like(h_ref)
        recv_buf[...] = jnp.zeros_like(recv_buf)
        bar = pltpu.get_barrier_semaphore()
        pltpu.semaphore_signal(bar, device_id=partner,
                               device_id_type=pltpu.DeviceIdType.LOGICAL)
        pltpu.semaphore_wait(bar, 1)
        for cp in copies:
            cp.wait()


def _epilogue(i, ngrid, send_buf, recv_buf, send_sem, recv_sem, partner):
    @pl.when(i == ngrid - 1)
    def _():
        _rcopy(send_buf, recv_buf, 0, send_sem, recv_sem, partner).wait_send()
        _rcopy(send_buf, recv_buf, 1, send_sem, recv_sem, partner).wait_send()
        _rcopy(send_buf, recv_buf, 0, send_sem, recv_sem, partner).wait_recv()


def _e1_body(tb, ngrid, hh):
    def body(myid_ref, x_ref, wih_ref, bih_ref, bhh_ref, wown_hbm, woth_hbm,
             ys_ref, wown_v, woth_v, send_buf, recv_buf, h_ref,
             semw1, semw2, send_sem, recv_sem):
        i = pl.program_id(0)
        partner = 1 - myid_ref[0]
        _prologue(i, partner, h_ref, recv_buf,
                  [(wown_hbm, wown_v, semw1), (woth_hbm, woth_v, semw2)])

        def step(t, c):
            s = i * tb + t
            xg_t = x_ref[pl.ds(t, 1), :] * wih_ref[...] + bih_ref[...]
            h_new = _step(s, xg_t, h_ref, wown_v, woth_v, bhh_ref,
                          send_buf, recv_buf, send_sem, recv_sem, partner, hh)
            ys_ref[pl.ds(t, 1), :] = h_new
            return c

        jax.lax.fori_loop(0, tb, step, 0)
        _epilogue(i, ngrid, send_buf, recv_buf, send_sem, recv_sem, partner)
    return body


def _run_e1(myid, x, wih_row, bih, bhh, w_own, w_oth, *, tb):
    seq = x.shape[0]
    hh = w_own.shape[0]
    g3 = w_own.shape[1]
    ngrid = seq // tb
    return pl.pallas_call(
        _e1_body(tb, ngrid, hh),
        grid=(ngrid,),
        in_specs=[
            pl.BlockSpec(memory_space=pltpu.SMEM),
            pl.BlockSpec((tb, 1), lambda i: (i, 0)),
            pl.BlockSpec((1, g3), lambda i: (0, 0)),
            pl.BlockSpec((1, g3), lambda i: (0, 0)),
            pl.BlockSpec((1, g3), lambda i: (0, 0)),
            pl.BlockSpec(memory_space=pl.ANY),
            pl.BlockSpec(memory_space=pl.ANY),
        ],
        out_specs=pl.BlockSpec((tb, hh), lambda i: (i, 0)),
        out_shape=jax.ShapeDtypeStruct((seq, hh), jnp.float32),
        scratch_shapes=[
            pltpu.VMEM((hh, g3), jnp.bfloat16),
            pltpu.VMEM((hh, g3), jnp.bfloat16),
            pltpu.VMEM((2, 1, hh), jnp.float32),
            pltpu.VMEM((2, 1, hh), jnp.float32),
            pltpu.VMEM((1, hh), jnp.float32),
            pltpu.SemaphoreType.DMA,
            pltpu.SemaphoreType.DMA,
            pltpu.SemaphoreType.DMA,
            pltpu.SemaphoreType.DMA,
        ],
        compiler_params=pltpu.CompilerParams(
            dimension_semantics=("arbitrary",),
            vmem_limit_bytes=_VMEM_LIMIT,
            collective_id=0,
        ),
        name="gru2_e1",
    )(myid, x, wih_row, bih, bhh, w_own, w_oth)


def _e2_body(tb, ngrid, hh):
    def body(myid_ref, ys1_ref, bih_ref, bhh_ref, wih_hbm, wown_hbm, woth_hbm,
             ht_ref, wih_v, wown_v, woth_v, xg_scr, send_buf, recv_buf, h_ref,
             semw1, semw2, semw3, send_sem, recv_sem):
        i = pl.program_id(0)
        partner = 1 - myid_ref[0]
        _prologue(i, partner, h_ref, recv_buf,
                  [(wih_hbm, wih_v, semw1), (wown_hbm, wown_v, semw2),
                   (woth_hbm, woth_v, semw3)])

        xg_scr[...] = jnp.dot(ys1_ref[...].astype(jnp.bfloat16), wih_v[...],
                              preferred_element_type=jnp.float32) + bih_ref[...]

        def step(t, c):
            s = i * tb + t
            _step(s, xg_scr[pl.ds(t, 1), :], h_ref, wown_v, woth_v, bhh_ref,
                  send_buf, recv_buf, send_sem, recv_sem, partner, hh)
            return c

        jax.lax.fori_loop(0, tb, step, 0)
        ht_ref[...] = h_ref[...]
        _epilogue(i, ngrid, send_buf, recv_buf, send_sem, recv_sem, partner)
    return body


def _run_e2(myid, ys1, bih, bhh, wih_t, w_own, w_oth, *, tb):
    seq, hid = ys1.shape
    hh = w_own.shape[0]
    g3 = w_own.shape[1]
    ngrid = seq // tb
    return pl.pallas_call(
        _e2_body(tb, ngrid, hh),
        grid=(ngrid,),
        in_specs=[
            pl.BlockSpec(memory_space=pltpu.SMEM),
            pl.BlockSpec((tb, hid), lambda i: (i, 0)),
            pl.BlockSpec((1, g3), lambda i: (0, 0)),
            pl.BlockSpec((1, g3), lambda i: (0, 0)),
            pl.BlockSpec(memory_space=pl.ANY),
            pl.BlockSpec(memory_space=pl.ANY),
            pl.BlockSpec(memory_space=pl.ANY),
        ],
        out_specs=pl.BlockSpec((1, hh), lambda i: (0, 0)),
        out_shape=jax.ShapeDtypeStruct((1, hh), jnp.float32),
        scratch_shapes=[
            pltpu.VMEM((hid, g3), jnp.bfloat16),
            pltpu.VMEM((hh, g3), jnp.bfloat16),
            pltpu.VMEM((hh, g3), jnp.bfloat16),
            pltpu.VMEM((tb, g3), jnp.float32),
            pltpu.VMEM((2, 1, hh), jnp.float32),
            pltpu.VMEM((2, 1, hh), jnp.float32),
            pltpu.VMEM((1, hh), jnp.float32),
            pltpu.SemaphoreType.DMA,
            pltpu.SemaphoreType.DMA,
            pltpu.SemaphoreType.DMA,
            pltpu.SemaphoreType.DMA,
            pltpu.SemaphoreType.DMA,
        ],
        compiler_params=pltpu.CompilerParams(
            dimension_semantics=("arbitrary",),
            vmem_limit_bytes=_VMEM_LIMIT,
            collective_id=1,
        ),
        name="gru2_e2",
    )(myid, ys1, bih, bhh, wih_t, w_own, w_oth)


def _d1_body(tb, ngrid, hh):
    def body(myid_ref, emb_ref, bih_ref, bhh_ref, wih_hbm, wown_hbm, woth_hbm,
             ys_ref, wih_v, wown_v, woth_v, xg_ref, send_buf, recv_buf, h_ref,
             semw1, semw2, semw3, send_sem, recv_sem):
        i = pl.program_id(0)
        partner = 1 - myid_ref[0]
        _prologue(i, partner, h_ref, recv_buf,
                  [(wih_hbm, wih_v, semw1), (wown_hbm, wown_v, semw2),
                   (woth_hbm, woth_v, semw3)])

        @pl.when(i == 0)
        def _():
            xg_ref[...] = jnp.dot(emb_ref[...].astype(jnp.bfloat16),
                                  wih_v[...],
                                  preferred_element_type=jnp.float32
                                  ) + bih_ref[...]

        def step(t, c):
            s = i * tb + t
            h_new = _step(s, xg_ref[...], h_ref, wown_v, woth_v, bhh_ref,
                          send_buf, recv_buf, send_sem, recv_sem, partner, hh)
            ys_ref[pl.ds(t, 1), :] = h_new
            return c

        jax.lax.fori_loop(0, tb, step, 0)
        _epilogue(i, ngrid, send_buf, recv_buf, send_sem, recv_sem, partner)
    return body


def _run_d1(myid, emb_vec, bih, bhh, wih_t, w_own, w_oth, *, seq, tb):
    emb = wih_t.shape[0]
    hh = w_own.shape[0]
    g3 = w_own.shape[1]
    ngrid = seq // tb
    return pl.pallas_call(
        _d1_body(tb, ngrid, hh),
        grid=(ngrid,),
        in_specs=[
            pl.BlockSpec(memory_space=pltpu.SMEM),
            pl.BlockSpec((1, emb), lambda i: (0, 0)),
            pl.BlockSpec((1, g3), lambda i: (0, 0)),
            pl.BlockSpec((1, g3), lambda i: (0, 0)),
            pl.BlockSpec(memory_space=pl.ANY),
            pl.BlockSpec(memory_space=pl.ANY),
            pl.BlockSpec(memory_space=pl.ANY),
        ],
        out_specs=pl.BlockSpec((tb, hh), lambda i: (i, 0)),
        out_shape=jax.ShapeDtypeStruct((seq, hh), jnp.float32),
        scratch_shapes=[
            pltpu.VMEM((emb, g3), jnp.bfloat16),
            pltpu.VMEM((hh, g3), jnp.bfloat16),
            pltpu.VMEM((hh, g3), jnp.bfloat16),
            pltpu.VMEM((1, g3), jnp.float32),
            pltpu.VMEM((2, 1, hh), jnp.float32),
            pltpu.VMEM((2, 1, hh), jnp.float32),
            pltpu.VMEM((1, hh), jnp.float32),
            pltpu.SemaphoreType.DMA,
            pltpu.SemaphoreType.DMA,
            pltpu.SemaphoreType.DMA,
            pltpu.SemaphoreType.DMA,
            pltpu.SemaphoreType.DMA,
        ],
        compiler_params=pltpu.CompilerParams(
            dimension_semantics=("arbitrary",),
            vmem_limit_bytes=_VMEM_LIMIT,
            collective_id=2,
        ),
        name="gru2_d1",
    )(myid, emb_vec, bih, bhh, wih_t, w_own, w_oth)


def _d2_body(tb, ngrid, hh):
    def body(myid_ref, ys2_ref, bih_ref, bhh_ref, outw_ref, outb_ref,
             wih_hbm, wown_hbm, woth_hbm,
             y_ref, wih_v, wown_v, woth_v, xg_scr, ys3_scr, send_buf,
             recv_buf, h_ref, semw1, semw2, semw3, send_sem, recv_sem):
        i = pl.program_id(0)
        partner = 1 - myid_ref[0]
        _prologue(i, partner, h_ref, recv_buf,
                  [(wih_hbm, wih_v, semw1), (wown_hbm, wown_v, semw2),
                   (woth_hbm, woth_v, semw3)])

        xg_scr[...] = jnp.dot(ys2_ref[...].astype(jnp.bfloat16), wih_v[...],
                              preferred_element_type=jnp.float32) + bih_ref[...]

        def step(t, c):
            s = i * tb + t
            h_new = _step(s, xg_scr[pl.ds(t, 1), :], h_ref, wown_v, woth_v,
                          bhh_ref, send_buf, recv_buf, send_sem, recv_sem,
                          partner, hh)
            ys3_scr[pl.ds(t, 1), :] = h_new
            return c

        jax.lax.fori_loop(0, tb, step, 0)
        y_ref[...] = jnp.dot(ys3_scr[...].astype(jnp.bfloat16), outw_ref[...],
                             preferred_element_type=jnp.float32) + outb_ref[...]
        _epilogue(i, ngrid, send_buf, recv_buf, send_sem, recv_sem, partner)
    return body


def _run_d2(myid, ys2, bih, bhh, out_wt, out_b_half, wih_t, w_own, w_oth, *, tb):
    seq, emb = ys2.shape
    hh = w_own.shape[0]
    g3 = w_own.shape[1]
    ngrid = seq // tb
    return pl.pallas_call(
        _d2_body(tb, ngrid, hh),
        grid=(ngrid,),
        in_specs=[
            pl.BlockSpec(memory_space=pltpu.SMEM),
            pl.BlockSpec((tb, emb), lambda i: (i, 0)),
            pl.BlockSpec((1, g3), lambda i: (0, 0)),
            pl.BlockSpec((1, g3), lambda i: (0, 0)),
            pl.BlockSpec((hh, 1), lambda i: (0, 0)),
            pl.BlockSpec((1, 1), lambda i: (0, 0)),
            pl.BlockSpec(memory_space=pl.ANY),
            pl.BlockSpec(memory_space=pl.ANY),
            pl.BlockSpec(memory_space=pl.ANY),
        ],
        out_specs=pl.BlockSpec((tb, 1), lambda i: (i, 0)),
        out_shape=jax.ShapeDtypeStruct((seq, 1), jnp.float32),
        scratch_shapes=[
            pltpu.VMEM((emb, g3), jnp.bfloat16),
            pltpu.VMEM((hh, g3), jnp.bfloat16),
            pltpu.VMEM((hh, g3), jnp.bfloat16),
            pltpu.VMEM((tb, g3), jnp.float32),
            pltpu.VMEM((tb, hh), jnp.float32),
            pltpu.VMEM((2, 1, hh), jnp.float32),
            pltpu.VMEM((2, 1, hh), jnp.float32),
            pltpu.VMEM((1, hh), jnp.float32),
            pltpu.SemaphoreType.DMA,
            pltpu.SemaphoreType.DMA,
            pltpu.SemaphoreType.DMA,
            pltpu.SemaphoreType.DMA,
            pltpu.SemaphoreType.DMA,
        ],
        compiler_params=pltpu.CompilerParams(
            dimension_semantics=("arbitrary",),
            vmem_limit_bytes=_VMEM_LIMIT,
            collective_id=3,
        ),
        name="gru2_d2",
    )(myid, ys2, bih, bhh, out_wt, out_b_half, wih_t, w_own, w_oth)


def _pack_cols(w_t, h):
    hh = h // 2
    parts = []
    for c in range(2):
        parts.append(jnp.concatenate(
            [w_t[:, c * hh:(c + 1) * hh],
             w_t[:, h + c * hh: h + (c + 1) * hh],
             w_t[:, 2 * h + c * hh: 2 * h + (c + 1) * hh]], axis=1))
    return jnp.stack(parts)


def _pack_whh(whh, h):
    hh = h // 2
    packed = _pack_cols(whh.T, h)
    w_own = jnp.stack([packed[0, :hh], packed[1, hh:]]).astype(jnp.bfloat16)
    w_oth = jnp.stack([packed[0, hh:], packed[1, :hh]]).astype(jnp.bfloat16)
    return w_own, w_oth


def _pack_bias(b, h):
    return _pack_cols(b.reshape(1, -1), h)


def _two_core_fn(seq, hid, emb, tb, tb2):
    def fn(x, wih1, bih1, bhh1, whh1_own, whh1_oth,
           wih2, bih2, bhh2, whh2_own, whh2_oth,
           wih3, bih3, bhh3, whh3_own, whh3_oth,
           wih4, bih4, bhh4, whh4_own, whh4_oth,
           outw, outb_half):
        myid = jax.lax.axis_index("c").reshape((1,)).astype(jnp.int32)
        sq = lambda a: a[0]

        ys1_c = _run_e1(myid, x, sq(wih1), sq(bih1), sq(bhh1),
                        sq(whh1_own), sq(whh1_oth), tb=tb)
        ys1 = jax.lax.all_gather(ys1_c, "c", axis=1, tiled=True)

        ht_c = _run_e2(myid, ys1, sq(bih2), sq(bhh2), sq(wih2),
                       sq(whh2_own), sq(whh2_oth), tb=tb)
        emb_full = jax.lax.all_gather(ht_c, "c", axis=1, tiled=True)

        ys2_c = _run_d1(myid, emb_full, sq(bih3), sq(bhh3), sq(wih3),
                        sq(whh3_own), sq(whh3_oth), seq=seq, tb=tb)
        ys2 = jax.lax.all_gather(ys2_c, "c", axis=1, tiled=True)

        y_part = _run_d2(myid, ys2, sq(bih4), sq(bhh4), sq(outw),
                         outb_half, sq(wih4), sq(whh4_own), sq(whh4_oth),
                         tb=tb2)
        return jax.lax.psum(y_part, "c")
    return fn


def kernel(x, e1_Wih, e1_Whh, e1_bih, e1_bhh,
           e2_Wih, e2_Whh, e2_bih, e2_bhh,
           d1_Wih, d1_Whh, d1_bih, d1_bhh,
           d2_Wih, d2_Whh, d2_bih, d2_bhh,
           out_W, out_b):
    bf16 = jnp.bfloat16
    seq = x.shape[0]
    hid = e1_Whh.shape[1]
    emb = e2_Whh.shape[1]
    hh, eh = hid // 2, emb // 2
    tb = min(512, seq)
    tb2 = min(256, seq)

    wih1 = _pack_cols(e1_Wih.T, hid)
    bih1, bhh1 = _pack_bias(e1_bih, hid), _pack_bias(e1_bhh, hid)
    whh1_own, whh1_oth = _pack_whh(e1_Whh, hid)

    wih2 = _pack_cols(e2_Wih.T, emb).astype(bf16)
    bih2, bhh2 = _pack_bias(e2_bih, emb), _pack_bias(e2_bhh, emb)
    whh2_own, whh2_oth = _pack_whh(e2_Whh, emb)

    wih3 = _pack_cols(d1_Wih.T, emb).astype(bf16)
    bih3, bhh3 = _pack_bias(d1_bih, emb), _pack_bias(d1_bhh, emb)
    whh3_own, whh3_oth = _pack_whh(d1_Whh, emb)

    wih4 = _pack_cols(d2_Wih.T, hid).astype(bf16)
    bih4, bhh4 = _pack_bias(d2_bih, hid), _pack_bias(d2_bhh, hid)
    whh4_own, whh4_oth = _pack_whh(d2_Whh, hid)

    outw = jnp.stack([out_W.T[:hh], out_W.T[hh:]]).astype(bf16)
    outb_half = (0.5 * out_b).reshape(1, 1).astype(jnp.float32)

    mesh = Mesh(np.array(jax.devices()[:2]), ("c",))
    shd = P("c")
    rep = P()
    fn = jax.shard_map(
        _two_core_fn(seq, hid, emb, tb, tb2),
        mesh=mesh,
        in_specs=(rep,
                  shd, shd, shd, shd, shd,
                  shd, shd, shd, shd, shd,
                  shd, shd, shd, shd, shd,
                  shd, shd, shd, shd, shd,
                  shd, rep),
        out_specs=rep,
        check_vma=False,
    )
    return fn(x.reshape(seq, 1),
              wih1, bih1, bhh1, whh1_own, whh1_oth,
              wih2, bih2, bhh2, whh2_own, whh2_oth,
              wih3, bih3, bhh3, whh3_own, whh3_oth,
              wih4, bih4, bhh4, whh4_own, whh4_oth,
              outw, outb_half)


# speedup vs baseline: 5.3600x; 1.1285x over previous
import functools

import jax
import jax.numpy as jnp
from jax.experimental import pallas as pl
from jax.experimental.pallas import tpu as pltpu
from jax.sharding import Mesh, PartitionSpec as P
import numpy as np

_VMEM_LIMIT = 56 * 1024 * 1024


def _gates(xg, hg, h, hh):
    r = jax.nn.sigmoid(xg[:, :hh] + hg[:, :hh])
    z = jax.nn.sigmoid(xg[:, hh:2 * hh] + hg[:, hh:2 * hh])
    n = jnp.tanh(xg[:, 2 * hh:] + r * hg[:, 2 * hh:])
    return (1.0 - z) * n + z * h


def _rcopy(send_buf, recv_buf, slot, send_sem, recv_sem, partner):
    return pltpu.make_async_remote_copy(
        send_buf.at[slot], recv_buf.at[slot], send_sem, recv_sem,
        device_id=partner, device_id_type=pltpu.DeviceIdType.LOGICAL)


def _step(s, xg_t, h_ref, w_own, w_oth, bhh_ref, send_buf, recv_buf,
          send_sem, recv_sem, partner, hh):
    slot = jnp.bitwise_and(s, 1)
    slot_prev = jnp.bitwise_and(s + 1, 1)

    hb = h_ref[...].astype(jnp.bfloat16)
    hg_a = jnp.dot(hb, w_own[...], preferred_element_type=jnp.float32)

    @pl.when(s > 0)
    def _():
        _rcopy(send_buf, recv_buf, slot_prev, send_sem, recv_sem,
               partner).wait_recv()

    h_oth = recv_buf[slot_prev].astype(jnp.bfloat16)
    hg_b = jnp.dot(h_oth, w_oth[...], preferred_element_type=jnp.float32)
    hg = hg_a + hg_b + bhh_ref[...]
    h_new = _gates(xg_t, hg, h_ref[...], hh)
    h_ref[...] = h_new

    @pl.when(s >= 2)
    def _():
        _rcopy(send_buf, recv_buf, slot, send_sem, recv_sem,
               partner).wait_send()

    send_buf[slot] = h_new
    _rcopy(send_buf, recv_buf, slot, send_sem, recv_sem, partner).start()
    return h_new


def _prologue(i, partner, h_ref, recv_buf, dma_pairs):
    @pl.when(i == 0)
    def _():
        copies = [pltpu.make_async_copy(src, dst, sem)
                  for src, dst, sem in dma_pairs]
        for cp in copies:
            cp.start()
        h_ref[...] = jnp.zeros_like(h_ref)
        recv_buf[...] = jnp.zeros_like(recv_buf)
        bar = pltpu.get_barrier_semaphore()
        pltpu.semaphore_signal(bar, device_id=partner,
                               device_id_type=pltpu.DeviceIdType.LOGICAL)
        pltpu.semaphore_wait(bar, 1)
        for cp in copies:
            cp.wait()


def _epilogue(i, ngrid, send_buf, recv_buf, send_sem, recv_sem, partner):
    @pl.when(i == ngrid - 1)
    def _():
        _rcopy(send_buf, recv_buf, 0, send_sem, recv_sem, partner).wait_send()
        _rcopy(send_buf, recv_buf, 1, send_sem, recv_sem, partner).wait_send()
        _rcopy(send_buf, recv_buf, 0, send_sem, recv_sem, partner).wait_recv()


def _e1_body(tb, ngrid, hh):
    def body(myid_ref, x_ref, wih_ref, bih_ref, bhh_ref, wown_hbm, woth_hbm,
             ys_ref, wown_v, woth_v, send_buf, recv_buf, h_ref,
             semw1, semw2, send_sem, recv_sem):
        i = pl.program_id(0)
        partner = 1 - myid_ref[0]
        _prologue(i, partner, h_ref, recv_buf,
                  [(wown_hbm, wown_v, semw1), (woth_hbm, woth_v, semw2)])

        def step(t, c):
            s = i * tb + t
            xg_t = x_ref[pl.ds(t, 1), :] * wih_ref[...] + bih_ref[...]
            h_new = _step(s, xg_t, h_ref, wown_v, woth_v, bhh_ref,
                          send_buf, recv_buf, send_sem, recv_sem, partner, hh)
            ys_ref[pl.ds(t, 1), :] = h_new
            return c

        jax.lax.fori_loop(0, tb, step, 0)
        _epilogue(i, ngrid, send_buf, recv_buf, send_sem, recv_sem, partner)
    return body


def _run_e1(myid, x, wih_row, bih, bhh, w_own, w_oth, *, tb):
    seq = x.shape[0]
    hh = w_own.shape[0]
    g3 = w_own.shape[1]
    ngrid = seq // tb
    return pl.pallas_call(
        _e1_body(tb, ngrid, hh),
        grid=(ngrid,),
        in_specs=[
            pl.BlockSpec(memory_space=pltpu.SMEM),
            pl.BlockSpec((tb, 1), lambda i: (i, 0)),
            pl.BlockSpec((1, g3), lambda i: (0, 0)),
            pl.BlockSpec((1, g3), lambda i: (0, 0)),
            pl.BlockSpec((1, g3), lambda i: (0, 0)),
            pl.BlockSpec(memory_space=pl.ANY),
            pl.BlockSpec(memory_space=pl.ANY),
        ],
        out_specs=pl.BlockSpec((tb, hh), lambda i: (i, 0)),
        out_shape=jax.ShapeDtypeStruct((seq, hh), jnp.float32),
        scratch_shapes=[
            pltpu.VMEM((hh, g3), jnp.bfloat16),
            pltpu.VMEM((hh, g3), jnp.bfloat16),
            pltpu.VMEM((2, 1, hh), jnp.float32),
            pltpu.VMEM((2, 1, hh), jnp.float32),
            pltpu.VMEM((1, hh), jnp.float32),
            pltpu.SemaphoreType.DMA,
            pltpu.SemaphoreType.DMA,
            pltpu.SemaphoreType.DMA,
            pltpu.SemaphoreType.DMA,
        ],
        compiler_params=pltpu.CompilerParams(
            dimension_semantics=("arbitrary",),
            vmem_limit_bytes=_VMEM_LIMIT,
            collective_id=0,
        ),
        name="gru2_e1",
    )(myid, x, wih_row, bih, bhh, w_own, w_oth)


def _e2l_body(tb, emb):
    def body(ys1_ref, bih_ref, bhh_ref, wih_hbm, whh_hbm, ht_ref,
             wih_v, whh_v, xg_scr, h_ref, sem1, sem2):
        @pl.when(pl.program_id(0) == 0)
        def _():
            cp1 = pltpu.make_async_copy(wih_hbm, wih_v, sem1)
            cp2 = pltpu.make_async_copy(whh_hbm, whh_v, sem2)
            cp1.start()
            cp2.start()
            cp1.wait()
            cp2.wait()
            h_ref[...] = jnp.zeros_like(h_ref)

        xg_scr[...] = jnp.dot(ys1_ref[...].astype(jnp.bfloat16), wih_v[...],
                              preferred_element_type=jnp.float32) + bih_ref[...]

        def step(t, c):
            hb = h_ref[...].astype(jnp.bfloat16)
            hg = jnp.dot(hb, whh_v[...],
                         preferred_element_type=jnp.float32) + bhh_ref[...]
            h_ref[...] = _gates(xg_scr[pl.ds(t, 1), :], hg, h_ref[...], emb)
            return c

        jax.lax.fori_loop(0, tb, step, 0)
        ht_ref[...] = h_ref[...]
    return body


def _run_e2l(ys1, bih, bhh, wih_t, whh_t, *, tb):
    seq, hid = ys1.shape
    emb = whh_t.shape[0]
    return pl.pallas_call(
        _e2l_body(tb, emb),
        grid=(seq // tb,),
        in_specs=[
            pl.BlockSpec((tb, hid), lambda i: (i, 0)),
            pl.BlockSpec((1, 3 * emb), lambda i: (0, 0)),
            pl.BlockSpec((1, 3 * emb), lambda i: (0, 0)),
            pl.BlockSpec(memory_space=pl.ANY),
            pl.BlockSpec(memory_space=pl.ANY),
        ],
        out_specs=pl.BlockSpec((1, emb), lambda i: (0, 0)),
        out_shape=jax.ShapeDtypeStruct((1, emb), jnp.float32),
        scratch_shapes=[
            pltpu.VMEM((hid, 3 * emb), jnp.bfloat16),
            pltpu.VMEM((emb, 3 * emb), jnp.bfloat16),
            pltpu.VMEM((tb, 3 * emb), jnp.float32),
            pltpu.VMEM((1, emb), jnp.float32),
            pltpu.SemaphoreType.DMA,
            pltpu.SemaphoreType.DMA,
        ],
        compiler_params=pltpu.CompilerParams(
            dimension_semantics=("arbitrary",),
            vmem_limit_bytes=_VMEM_LIMIT,
        ),
        name="gru_e2l",
    )(ys1, bih, bhh, wih_t, whh_t)


def _d1l_body(tb, emb):
    def body(emb_ref, bih_ref, bhh_ref, wih_hbm, whh_hbm, ys_ref,
             wih_v, whh_v, xg_ref, h_ref, sem1, sem2):
        @pl.when(pl.program_id(0) == 0)
        def _():
            cp1 = pltpu.make_async_copy(wih_hbm, wih_v, sem1)
            cp2 = pltpu.make_async_copy(whh_hbm, whh_v, sem2)
            cp1.start()
            cp2.start()
            cp1.wait()
            cp2.wait()
            h_ref[...] = jnp.zeros_like(h_ref)
            xg_ref[...] = jnp.dot(emb_ref[...].astype(jnp.bfloat16),
                                  wih_v[...],
                                  preferred_element_type=jnp.float32
                                  ) + bih_ref[...]

        def step(t, c):
            hb = h_ref[...].astype(jnp.bfloat16)
            hg = jnp.dot(hb, whh_v[...],
                         preferred_element_type=jnp.float32) + bhh_ref[...]
            h_new = _gates(xg_ref[...], hg, h_ref[...], emb)
            h_ref[...] = h_new
            ys_ref[pl.ds(t, 1), :] = h_new
            return c

        jax.lax.fori_loop(0, tb, step, 0)
    return body


def _run_d1l(emb_vec, bih, bhh, wih_t, whh_t, *, seq, tb):
    emb = whh_t.shape[0]
    return pl.pallas_call(
        _d1l_body(tb, emb),
        grid=(seq // tb,),
        in_specs=[
            pl.BlockSpec((1, emb), lambda i: (0, 0)),
            pl.BlockSpec((1, 3 * emb), lambda i: (0, 0)),
            pl.BlockSpec((1, 3 * emb), lambda i: (0, 0)),
            pl.BlockSpec(memory_space=pl.ANY),
            pl.BlockSpec(memory_space=pl.ANY),
        ],
        out_specs=pl.BlockSpec((tb, emb), lambda i: (i, 0)),
        out_shape=jax.ShapeDtypeStruct((seq, emb), jnp.float32),
        scratch_shapes=[
            pltpu.VMEM((emb, 3 * emb), jnp.bfloat16),
            pltpu.VMEM((emb, 3 * emb), jnp.bfloat16),
            pltpu.VMEM((1, 3 * emb), jnp.float32),
            pltpu.VMEM((1, emb), jnp.float32),
            pltpu.SemaphoreType.DMA,
            pltpu.SemaphoreType.DMA,
        ],
        compiler_params=pltpu.CompilerParams(
            dimension_semantics=("arbitrary",),
            vmem_limit_bytes=_VMEM_LIMIT,
        ),
        name="gru_d1l",
    )(emb_vec, bih, bhh, wih_t, whh_t)


def _d2_body(tb, ngrid, hh):
    def body(myid_ref, ys2_ref, bih_ref, bhh_ref, outw_ref, outb_ref,
             wih_hbm, wown_hbm, woth_hbm,
             y_ref, wih_v, wown_v, woth_v, xg_scr, ys3_scr, send_buf,
             recv_buf, h_ref, semw1, semw2, semw3, send_sem, recv_sem):
        i = pl.program_id(0)
        partner = 1 - myid_ref[0]
        _prologue(i, partner, h_ref, recv_buf,
                  [(wih_hbm, wih_v, semw1), (wown_hbm, wown_v, semw2),
                   (woth_hbm, woth_v, semw3)])

        xg_scr[...] = jnp.dot(ys2_ref[...].astype(jnp.bfloat16), wih_v[...],
                              preferred_element_type=jnp.float32) + bih_ref[...]

        def step(t, c):
            s = i * tb + t
            h_new = _step(s, xg_scr[pl.ds(t, 1), :], h_ref, wown_v, woth_v,
                          bhh_ref, send_buf, recv_buf, send_sem, recv_sem,
                          partner, hh)
            ys3_scr[pl.ds(t, 1), :] = h_new
            return c

        jax.lax.fori_loop(0, tb, step, 0)
        y_ref[...] = jnp.dot(ys3_scr[...].astype(jnp.bfloat16), outw_ref[...],
                             preferred_element_type=jnp.float32) + outb_ref[...]
        _epilogue(i, ngrid, send_buf, recv_buf, send_sem, recv_sem, partner)
    return body


def _run_d2(myid, ys2, bih, bhh, out_wt, out_b_half, wih_t, w_own, w_oth, *, tb):
    seq, emb = ys2.shape
    hh = w_own.shape[0]
    g3 = w_own.shape[1]
    ngrid = seq // tb
    return pl.pallas_call(
        _d2_body(tb, ngrid, hh),
        grid=(ngrid,),
        in_specs=[
            pl.BlockSpec(memory_space=pltpu.SMEM),
            pl.BlockSpec((tb, emb), lambda i: (i, 0)),
            pl.BlockSpec((1, g3), lambda i: (0, 0)),
            pl.BlockSpec((1, g3), lambda i: (0, 0)),
            pl.BlockSpec((hh, 1), lambda i: (0, 0)),
            pl.BlockSpec((1, 1), lambda i: (0, 0)),
            pl.BlockSpec(memory_space=pl.ANY),
            pl.BlockSpec(memory_space=pl.ANY),
            pl.BlockSpec(memory_space=pl.ANY),
        ],
        out_specs=pl.BlockSpec((tb, 1), lambda i: (i, 0)),
        out_shape=jax.ShapeDtypeStruct((seq, 1), jnp.float32),
        scratch_shapes=[
            pltpu.VMEM((emb, g3), jnp.bfloat16),
            pltpu.VMEM((hh, g3), jnp.bfloat16),
            pltpu.VMEM((hh, g3), jnp.bfloat16),
            pltpu.VMEM((tb, g3), jnp.float32),
            pltpu.VMEM((tb, hh), jnp.float32),
            pltpu.VMEM((2, 1, hh), jnp.float32),
            pltpu.VMEM((2, 1, hh), jnp.float32),
            pltpu.VMEM((1, hh), jnp.float32),
            pltpu.SemaphoreType.DMA,
            pltpu.SemaphoreType.DMA,
            pltpu.SemaphoreType.DMA,
            pltpu.SemaphoreType.DMA,
            pltpu.SemaphoreType.DMA,
        ],
        compiler_params=pltpu.CompilerParams(
            dimension_semantics=("arbitrary",),
            vmem_limit_bytes=_VMEM_LIMIT,
            collective_id=3,
        ),
        name="gru2_d2",
    )(myid, ys2, bih, bhh, out_wt, out_b_half, wih_t, w_own, w_oth)


def _pack_cols(w_t, h):
    hh = h // 2
    parts = []
    for c in range(2):
        parts.append(jnp.concatenate(
            [w_t[:, c * hh:(c + 1) * hh],
             w_t[:, h + c * hh: h + (c + 1) * hh],
             w_t[:, 2 * h + c * hh: 2 * h + (c + 1) * hh]], axis=1))
    return jnp.stack(parts)


def _pack_whh(whh, h):
    hh = h // 2
    packed = _pack_cols(whh.T, h)
    w_own = jnp.stack([packed[0, :hh], packed[1, hh:]]).astype(jnp.bfloat16)
    w_oth = jnp.stack([packed[0, hh:], packed[1, :hh]]).astype(jnp.bfloat16)
    return w_own, w_oth


def _pack_bias(b, h):
    return _pack_cols(b.reshape(1, -1), h)


def _two_core_fn(seq, hid, emb, tb, tb2):
    def fn(x, wih1, bih1, bhh1, whh1_own, whh1_oth,
           wih2, bih2, bhh2, whh2,
           wih3, bih3, bhh3, whh3,
           wih4, bih4, bhh4, whh4_own, whh4_oth,
           outw, outb_half):
        myid = jax.lax.axis_index("c").reshape((1,)).astype(jnp.int32)
        sq = lambda a: a[0]

        ys1_c = _run_e1(myid, x, sq(wih1), sq(bih1), sq(bhh1),
                        sq(whh1_own), sq(whh1_oth), tb=tb)
        ys1 = jax.lax.all_gather(ys1_c, "c", axis=1, tiled=True)

        emb_full = _run_e2l(ys1, bih2, bhh2, wih2, whh2, tb=tb)
        ys2 = _run_d1l(emb_full, bih3, bhh3, wih3, whh3, seq=seq, tb=tb)

        y_part = _run_d2(myid, ys2, sq(bih4), sq(bhh4), sq(outw),
                         outb_half, sq(wih4), sq(whh4_own), sq(whh4_oth),
                         tb=tb2)
        return jax.lax.psum(y_part, "c")
    return fn


def kernel(x, e1_Wih, e1_Whh, e1_bih, e1_bhh,
           e2_Wih, e2_Whh, e2_bih, e2_bhh,
           d1_Wih, d1_Whh, d1_bih, d1_bhh,
           d2_Wih, d2_Whh, d2_bih, d2_bhh,
           out_W, out_b):
    bf16 = jnp.bfloat16
    seq = x.shape[0]
    hid = e1_Whh.shape[1]
    emb = e2_Whh.shape[1]
    hh, eh = hid // 2, emb // 2
    tb = min(512, seq)
    tb2 = min(256, seq)

    wih1 = _pack_cols(e1_Wih.T, hid)
    bih1, bhh1 = _pack_bias(e1_bih, hid), _pack_bias(e1_bhh, hid)
    whh1_own, whh1_oth = _pack_whh(e1_Whh, hid)

    wih2 = e2_Wih.T.astype(bf16)
    bih2, bhh2 = e2_bih.reshape(1, -1), e2_bhh.reshape(1, -1)
    whh2 = e2_Whh.T.astype(bf16)

    wih3 = d1_Wih.T.astype(bf16)
    bih3, bhh3 = d1_bih.reshape(1, -1), d1_bhh.reshape(1, -1)
    whh3 = d1_Whh.T.astype(bf16)

    wih4 = _pack_cols(d2_Wih.T, hid).astype(bf16)
    bih4, bhh4 = _pack_bias(d2_bih, hid), _pack_bias(d2_bhh, hid)
    whh4_own, whh4_oth = _pack_whh(d2_Whh, hid)

    outw = jnp.stack([out_W.T[:hh], out_W.T[hh:]]).astype(bf16)
    outb_half = (0.5 * out_b).reshape(1, 1).astype(jnp.float32)

    mesh = Mesh(np.array(jax.devices()[:2]), ("c",))
    shd = P("c")
    rep = P()
    fn = jax.shard_map(
        _two_core_fn(seq, hid, emb, tb, tb2),
        mesh=mesh,
        in_specs=(rep,
                  shd, shd, shd, shd, shd,
                  rep, rep, rep, rep,
                  rep, rep, rep, rep,
                  shd, shd, shd, shd, shd,
                  shd, rep),
        out_specs=rep,
        check_vma=False,
    )
    return fn(x.reshape(seq, 1),
              wih1, bih1, bhh1, whh1_own, whh1_oth,
              wih2, bih2, bhh2, whh2,
              wih3, bih3, bhh3, whh3,
              wih4, bih4, bhh4, whh4_own, whh4_oth,
              outw, outb_half)
